# baseline (speedup 1.0000x reference)
import functools

import jax
import jax.numpy as jnp
from jax import lax
from jax.experimental import pallas as pl
from jax.experimental.pallas import tpu as pltpu
from jax.experimental.pallas import tpu_sc as plsc

B = 128
N = 32768
NC = 2
NS = 16
NW = NC * NS
ROWS_PER_W = B // NW
C = 8192
NCH = N // C
L = 16
VPC = C // L


def _lif_body(x_hbm, z_hbm, v_hbm, i_hbm, xb, zb, ib, vconst, patch):
    wid = lax.axis_index("s") * NC + lax.axis_index("c")

    def fill(j, _):
        vconst[pl.ds(j * L, L)] = jnp.full((L,), -5.0, jnp.float32)
        return 0
    lax.fori_loop(0, VPC, fill, 0)

    for r in range(ROWS_PER_W):
        row = wid * ROWS_PER_W + r
        mx = jnp.full((L,), -jnp.inf, jnp.float32)
        mi = jnp.zeros((L,), jnp.int32)
        for c in range(NCH):
            pltpu.sync_copy(x_hbm.at[row, pl.ds(c * C, C)], xb)

            def step(t, carry, _c=c):
                mxc, mic = carry
                off = t * L
                xv = xb[pl.ds(off, L)]
                iv = xv * 0.5
                vv = iv * 0.5
                spk = vv >= 1.0
                zb[pl.ds(off, L)] = jnp.where(spk, jnp.float32(1.0),
                                              jnp.float32(0.0))
                ib[pl.ds(off, L)] = iv
                idx = lax.iota(jnp.int32, L) + (_c * C + off)
                upd = xv > mxc
                mxc = jnp.where(upd, xv, mxc)
                mic = jnp.where(upd, idx, mic)
                return mxc, mic

            mx, mi = lax.fori_loop(0, VPC, step, (mx, mi))
            pltpu.sync_copy(zb, z_hbm.at[row, pl.ds(c * C, C)])
            pltpu.sync_copy(ib, i_hbm.at[row, pl.ds(c * C, C)])

        m = jnp.float32(-jnp.inf)
        win = jnp.int32(2**31 - 1)
        for j in range(L):
            a = mx[j]
            bidx = mi[j]
            better = jnp.logical_or(a > m,
                                    jnp.logical_and(a == m, bidx < win))
            m = jnp.where(better, a, m)
            win = jnp.where(better, bidx, win)
        any_spike = m >= 4.0

        @pl.when(any_spike)
        def _():
            base = (win // L) * L
            off = win - base
            patch[...] = jnp.where(lax.iota(jnp.int32, L) == off,
                                   jnp.float32(0.0), jnp.float32(-5.0))
            for c in range(NCH):
                pltpu.sync_copy(vconst, v_hbm.at[row, pl.ds(c * C, C)])
            pltpu.sync_copy(patch, v_hbm.at[row, pl.ds(base, L)])

        @pl.when(jnp.logical_not(any_spike))
        def _():
            for c in range(NCH):
                pltpu.sync_copy(x_hbm.at[row, pl.ds(c * C, C)], xb)

                def vstep(t, _):
                    off = t * L
                    xb[pl.ds(off, L)] = xb[pl.ds(off, L)] * 0.25
                    return 0

                lax.fori_loop(0, VPC, vstep, 0)
                pltpu.sync_copy(xb, v_hbm.at[row, pl.ds(c * C, C)])


@jax.jit
def _lif_sc(x):
    f32 = jnp.float32
    out = jax.ShapeDtypeStruct((B, N), f32)
    k = functools.partial(
        pl.kernel,
        mesh=plsc.VectorSubcoreMesh(core_axis_name="c", subcore_axis_name="s"),
        out_type=[out, out, out],
        scratch_types=[
            pltpu.VMEM((C,), f32),
            pltpu.VMEM((C,), f32),
            pltpu.VMEM((C,), f32),
            pltpu.VMEM((C,), f32),
            pltpu.VMEM((L,), f32),
        ],
    )(_lif_body)
    return k(x)


def kernel(x):
    z, new_v, i_new = _lif_sc(x)
    return z, new_v, i_new


# speedup vs baseline: 1.0900x; 1.0900x over previous
import functools

import jax
import jax.numpy as jnp
from jax import lax
from jax.experimental import pallas as pl
from jax.experimental.pallas import tpu as pltpu
from jax.experimental.pallas import tpu_sc as plsc

B = 128
N = 32768
NC = 2
NS = 16
NW = NC * NS
ROWS_PER_W = B // NW
C = 8192
NCH = N // C
TOT = ROWS_PER_W * NCH
L = 16
VPC = C // L


def _lif_body(x_hbm, z_hbm, v_hbm, i_hbm, xb, zb, ib, vconst, fb,
              semx, semo):
    wid = lax.axis_index("s") * NC + lax.axis_index("c")
    r0 = wid * ROWS_PER_W

    def fill(j, _):
        vconst[pl.ds(j * L, L)] = jnp.full((L,), -5.0, jnp.float32)
        return 0
    lax.fori_loop(0, N // L, fill, 0, unroll=8)

    hx = {}

    def issue_x(k):
        r, c = divmod(k, NCH)
        hx[k] = pltpu.async_copy(
            x_hbm.at[r0 + r, pl.ds(c * C, C)], xb.at[k % 2], semx.at[k % 2])

    ho = {}
    issue_x(0)
    for r in range(ROWS_PER_W):
        row = r0 + r
        mx = jnp.full((L,), -jnp.inf, jnp.float32)
        mi = jnp.zeros((L,), jnp.int32)
        for c in range(NCH):
            k = r * NCH + c
            b = k % 2
            if k + 1 < TOT:
                issue_x(k + 1)
            hx[k].wait()
            if k >= 2:
                for h in ho[k - 2]:
                    h.wait()
            xcb, zcb, icb = xb.at[b], zb.at[b], ib.at[b]

            def step(t, carry, _c=c, _xcb=xcb, _zcb=zcb, _icb=icb):
                mxc, mic = carry
                off = t * L
                xv = _xcb[pl.ds(off, L)]
                iv = xv * 0.5
                vv = iv * 0.5
                spk = vv >= 1.0
                _zcb[pl.ds(off, L)] = jnp.where(spk, jnp.float32(1.0),
                                                jnp.float32(0.0))
                _icb[pl.ds(off, L)] = iv
                idx = lax.iota(jnp.int32, L) + (_c * C + off)
                upd = xv > mxc
                mxc = jnp.where(upd, xv, mxc)
                mic = jnp.where(upd, idx, mic)
                return mxc, mic

            mx, mi = lax.fori_loop(0, VPC, step, (mx, mi), unroll=8)
            ho[k] = [
                pltpu.async_copy(zcb, z_hbm.at[row, pl.ds(c * C, C)],
                                 semo.at[b]),
                pltpu.async_copy(icb, i_hbm.at[row, pl.ds(c * C, C)],
                                 semo.at[b]),
            ]

        m = jnp.float32(-jnp.inf)
        win = jnp.int32(2**31 - 1)
        for j in range(L):
            a = mx[j]
            bidx = mi[j]
            better = jnp.logical_or(a > m,
                                    jnp.logical_and(a == m, bidx < win))
            m = jnp.where(better, a, m)
            win = jnp.where(better, bidx, win)
        any_spike = m >= 4.0

        @pl.when(any_spike)
        def _():
            base = (win // L) * L
            off = win - base
            vconst[pl.ds(base, L)] = jnp.where(
                lax.iota(jnp.int32, L) == off,
                jnp.float32(0.0), jnp.float32(-5.0))
            pltpu.sync_copy(vconst, v_hbm.at[row])
            vconst[pl.ds(base, L)] = jnp.full((L,), -5.0, jnp.float32)

        @pl.when(jnp.logical_not(any_spike))
        def _():
            for c in range(NCH):
                pltpu.sync_copy(x_hbm.at[row, pl.ds(c * C, C)], fb)

                def vstep(t, _):
                    off = t * L
                    fb[pl.ds(off, L)] = fb[pl.ds(off, L)] * 0.25
                    return 0

                lax.fori_loop(0, VPC, vstep, 0, unroll=8)
                pltpu.sync_copy(fb, v_hbm.at[row, pl.ds(c * C, C)])

    for k in (TOT - 2, TOT - 1):
        for h in ho[k]:
            h.wait()


@jax.jit
def _lif_sc(x):
    f32 = jnp.float32
    out = jax.ShapeDtypeStruct((B, N), f32)
    k = functools.partial(
        pl.kernel,
        mesh=plsc.VectorSubcoreMesh(core_axis_name="c", subcore_axis_name="s"),
        out_type=[out, out, out],
        scratch_types=[
            pltpu.VMEM((2, C), f32),
            pltpu.VMEM((2, C), f32),
            pltpu.VMEM((2, C), f32),
            pltpu.VMEM((N,), f32),
            pltpu.VMEM((C,), f32),
            pltpu.SemaphoreType.DMA((2,)),
            pltpu.SemaphoreType.DMA((2,)),
        ],
    )(_lif_body)
    return k(x)


def kernel(x):
    z, new_v, i_new = _lif_sc(x)
    return z, new_v, i_new


# speedup vs baseline: 1.6794x; 1.5407x over previous
import functools

import jax
import jax.numpy as jnp
from jax import lax
from jax.experimental import pallas as pl
from jax.experimental.pallas import tpu as pltpu
from jax.experimental.pallas import tpu_sc as plsc

B = 128
N = 32768
NC = 2
NS = 16
NW = NC * NS
ROWS_PER_W = B // NW
C = 8192
NCH = N // C
TOT = ROWS_PER_W * NCH
L = 16
U = 8
VPC = C // L
BIG = 2**31 - 1


def _lif_body(x_hbm, z_hbm, v_hbm, i_hbm, xb, zb, ib, vconst, fb,
              semx, semo):
    wid = lax.axis_index("s") * NC + lax.axis_index("c")
    r0 = wid * ROWS_PER_W
    ii = lax.iota(jnp.int32, L)

    def fill(j, _):
        vconst[pl.ds(j * L, L)] = jnp.full((L,), -5.0, jnp.float32)
        return 0
    lax.fori_loop(0, N // L, fill, 0, unroll=8)

    hx = {}

    def issue_x(k):
        r, c = divmod(k, NCH)
        hx[k] = pltpu.async_copy(
            x_hbm.at[r0 + r, pl.ds(c * C, C)], xb.at[k % 2], semx.at[k % 2])

    ho = {}
    issue_x(0)
    for r in range(ROWS_PER_W):
        row = r0 + r
        cmx = []
        for c in range(NCH):
            k = r * NCH + c
            b = k % 2
            if k + 1 < TOT:
                issue_x(k + 1)
            hx[k].wait()
            if k >= 2:
                for h in ho[k - 2]:
                    h.wait()
            xcb, zcb, icb = xb.at[b], zb.at[b], ib.at[b]

            def step(t, mxc, _xcb=xcb, _zcb=zcb, _icb=icb):
                off0 = t * (L * U)
                xs = [_xcb[pl.ds(off0 + u * L, L)] for u in range(U)]
                for u in range(U):
                    iv = xs[u] * 0.5
                    _icb[pl.ds(off0 + u * L, L)] = iv
                    spk = xs[u] >= 4.0
                    _zcb[pl.ds(off0 + u * L, L)] = jnp.where(
                        spk, jnp.float32(1.0), jnp.float32(0.0))
                m01 = jnp.maximum(xs[0], xs[1])
                m23 = jnp.maximum(xs[2], xs[3])
                m45 = jnp.maximum(xs[4], xs[5])
                m67 = jnp.maximum(xs[6], xs[7])
                mg = jnp.maximum(jnp.maximum(m01, m23),
                                 jnp.maximum(m45, m67))
                return jnp.maximum(mxc, mg)

            mxc = lax.fori_loop(0, VPC // U, step,
                                jnp.full((L,), -jnp.inf, jnp.float32),
                                unroll=2)
            cmx.append(mxc)
            ho[k] = [
                pltpu.async_copy(zcb, z_hbm.at[row, pl.ds(c * C, C)],
                                 semo.at[b]),
                pltpu.async_copy(icb, i_hbm.at[row, pl.ds(c * C, C)],
                                 semo.at[b]),
            ]

        cms = []
        for c in range(NCH):
            mc = cmx[c][0]
            for j in range(1, L):
                mc = jnp.maximum(mc, cmx[c][j])
            cms.append(mc)
        m = cms[0]
        for c in range(1, NCH):
            m = jnp.maximum(m, cms[c])
        cw = jnp.int32(NCH - 1)
        for c in range(NCH - 2, -1, -1):
            cw = jnp.where(cms[c] == m, jnp.int32(c), cw)
        any_spike = m >= 4.0

        @pl.when(any_spike)
        def _():
            pltpu.sync_copy(x_hbm.at[row, pl.ds(cw * C, C)], fb)
            mvec = lax.broadcast_in_dim(m, (L,), ())

            def sstep(t, fnd):
                xv = fb[pl.ds(t * L, L)]
                hit = jnp.logical_and(xv == mvec, fnd == BIG)
                return jnp.where(hit, ii + t * L, fnd)

            fnd = lax.fori_loop(0, VPC, sstep,
                                jnp.full((L,), BIG, jnp.int32),
                                unroll=8)
            win_in = fnd[0]
            for j in range(1, L):
                win_in = jnp.minimum(win_in, fnd[j])
            win = cw * C + win_in

            base = (win // L) * L
            off = win - base
            vconst[pl.ds(base, L)] = jnp.where(
                ii == off, jnp.float32(0.0), jnp.float32(-5.0))
            pltpu.sync_copy(vconst, v_hbm.at[row])
            vconst[pl.ds(base, L)] = jnp.full((L,), -5.0, jnp.float32)

        @pl.when(jnp.logical_not(any_spike))
        def _():
            for c in range(NCH):
                pltpu.sync_copy(x_hbm.at[row, pl.ds(c * C, C)], fb)

                def vstep(t, _):
                    off = t * L
                    fb[pl.ds(off, L)] = fb[pl.ds(off, L)] * 0.25
                    return 0

                lax.fori_loop(0, VPC, vstep, 0, unroll=8)
                pltpu.sync_copy(fb, v_hbm.at[row, pl.ds(c * C, C)])

    for k in (TOT - 2, TOT - 1):
        for h in ho[k]:
            h.wait()


@jax.jit
def _lif_sc(x):
    f32 = jnp.float32
    out = jax.ShapeDtypeStruct((B, N), f32)
    k = functools.partial(
        pl.kernel,
        mesh=plsc.VectorSubcoreMesh(core_axis_name="c", subcore_axis_name="s"),
        out_type=[out, out, out],
        scratch_types=[
            pltpu.VMEM((2, C), f32),
            pltpu.VMEM((2, C), f32),
            pltpu.VMEM((2, C), f32),
            pltpu.VMEM((N,), f32),
            pltpu.VMEM((C,), f32),
            pltpu.SemaphoreType.DMA((2,)),
            pltpu.SemaphoreType.DMA((2,)),
        ],
    )(_lif_body)
    return k(x)


def kernel(x):
    z, new_v, i_new = _lif_sc(x)
    return z, new_v, i_new


# speedup vs baseline: 2.4134x; 1.4371x over previous
import functools

import jax
import jax.numpy as jnp
from jax import lax
from jax.experimental import pallas as pl
from jax.experimental.pallas import tpu as pltpu
from jax.experimental.pallas import tpu_sc as plsc

B = 128
N = 32768
NC = 2
NS = 16
NW = NC * NS
ROWS_PER_W = B // NW
L = 16
U = 8
NSEG = 4
SEG = N // NSEG
VPS = SEG // L
FBC = 8192
BIG = 2**31 - 1


def _v_body(x_hbm, v_hbm, xb, vconst, fb, semx):
    wid = lax.axis_index("s") * NC + lax.axis_index("c")
    r0 = wid * ROWS_PER_W
    ii = lax.iota(jnp.int32, L)

    def fill(j, _):
        vconst[pl.ds(j * L, L)] = jnp.full((L,), -5.0, jnp.float32)
        return 0
    lax.fori_loop(0, N // L, fill, 0, unroll=8)

    hx = {}

    def issue_x(r):
        hx[r] = pltpu.async_copy(x_hbm.at[r0 + r], xb.at[r % 2],
                                 semx.at[r % 2])

    issue_x(0)
    for r in range(ROWS_PER_W):
        row = r0 + r
        b = r % 2
        if r + 1 < ROWS_PER_W:
            issue_x(r + 1)
        hx[r].wait()
        xrow = xb.at[b]

        smx = []
        for s in range(NSEG):
            def step(t, mxc, _s=s):
                off0 = _s * SEG + t * (L * U)
                xs = [xrow[pl.ds(off0 + u * L, L)] for u in range(U)]
                m01 = jnp.maximum(xs[0], xs[1])
                m23 = jnp.maximum(xs[2], xs[3])
                m45 = jnp.maximum(xs[4], xs[5])
                m67 = jnp.maximum(xs[6], xs[7])
                mg = jnp.maximum(jnp.maximum(m01, m23),
                                 jnp.maximum(m45, m67))
                return jnp.maximum(mxc, mg)

            smx.append(lax.fori_loop(
                0, VPS // U, step,
                jnp.full((L,), -jnp.inf, jnp.float32), unroll=2))

        sms = []
        for s in range(NSEG):
            mc = smx[s][0]
            for j in range(1, L):
                mc = jnp.maximum(mc, smx[s][j])
            sms.append(mc)
        m = sms[0]
        for s in range(1, NSEG):
            m = jnp.maximum(m, sms[s])
        sw = jnp.int32(NSEG - 1)
        for s in range(NSEG - 2, -1, -1):
            sw = jnp.where(sms[s] == m, jnp.int32(s), sw)
        any_spike = m >= 4.0

        @pl.when(any_spike)
        def _():
            mvec = lax.broadcast_in_dim(m, (L,), ())
            seg0 = sw * SEG

            def sstep(t, fnd):
                xv = xrow[pl.ds(seg0 + t * L, L)]
                hit = jnp.logical_and(xv == mvec, fnd == BIG)
                return jnp.where(hit, ii + t * L, fnd)

            fnd = lax.fori_loop(0, VPS, sstep,
                                jnp.full((L,), BIG, jnp.int32), unroll=8)
            win_in = fnd[0]
            for j in range(1, L):
                win_in = jnp.minimum(win_in, fnd[j])
            win = seg0 + win_in

            base = (win // L) * L
            off = win - base
            vconst[pl.ds(base, L)] = jnp.where(
                ii == off, jnp.float32(0.0), jnp.float32(-5.0))
            pltpu.sync_copy(vconst, v_hbm.at[row])
            vconst[pl.ds(base, L)] = jnp.full((L,), -5.0, jnp.float32)

        @pl.when(jnp.logical_not(any_spike))
        def _():
            for c in range(N // FBC):
                def vstep(t, _, _c=c):
                    fb[pl.ds(t * L, L)] = (
                        xrow[pl.ds(_c * FBC + t * L, L)] * 0.25)
                    return 0

                lax.fori_loop(0, FBC // L, vstep, 0, unroll=8)
                pltpu.sync_copy(fb, v_hbm.at[row, pl.ds(c * FBC, FBC)])


@jax.jit
def _lif_hybrid(x):
    f32 = jnp.float32

    def zi_body(x_ref, z_ref, i_ref):
        xv = x_ref[...]
        i_ref[...] = xv * 0.5
        z_ref[...] = jnp.where(xv >= 4.0, jnp.float32(1.0),
                               jnp.float32(0.0))

    z, i = pl.pallas_call(
        zi_body,
        grid=(16,),
        in_specs=[pl.BlockSpec((8, N), lambda g: (g, 0))],
        out_specs=[pl.BlockSpec((8, N), lambda g: (g, 0)),
                   pl.BlockSpec((8, N), lambda g: (g, 0))],
        out_shape=[jax.ShapeDtypeStruct((B, N), f32),
                   jax.ShapeDtypeStruct((B, N), f32)],
    )(x)

    v = functools.partial(
        pl.kernel,
        mesh=plsc.VectorSubcoreMesh(core_axis_name="c", subcore_axis_name="s"),
        out_type=jax.ShapeDtypeStruct((B, N), f32),
        scratch_types=[
            pltpu.VMEM((2, N), f32),
            pltpu.VMEM((N,), f32),
            pltpu.VMEM((FBC,), f32),
            pltpu.SemaphoreType.DMA((2,)),
        ],
    )(_v_body)(x)

    return z, v, i


def kernel(x):
    z, new_v, i_new = _lif_hybrid(x)
    return z, new_v, i_new


# speedup vs baseline: 2.6469x; 1.0967x over previous
import functools

import jax
import jax.numpy as jnp
from jax import lax
from jax.experimental import pallas as pl
from jax.experimental.pallas import tpu as pltpu
from jax.experimental.pallas import tpu_sc as plsc

B = 128
N = 32768
NC = 2
NS = 16
NW = NC * NS
ROWS_PER_W = B // NW
L = 16
U = 8
NSEG = 4
SEG = N // NSEG
VPS = SEG // L
FBC = 8192
BIG = 2**31 - 1


def _v_body(x_hbm, v_hbm, xb, vconst, semx, semv):
    wid = lax.axis_index("s") * NC + lax.axis_index("c")
    r0 = wid * ROWS_PER_W
    ii = lax.iota(jnp.int32, L)

    def refill(j, _):
        vconst[pl.ds(j * L, L)] = jnp.full((L,), -5.0, jnp.float32)
        return 0

    lax.fori_loop(0, N // L, refill, 0, unroll=8)

    hx = {}

    def issue_x(r):
        hx[r] = pltpu.async_copy(x_hbm.at[r0 + r], xb.at[r % 2],
                                 semx.at[r % 2])

    issue_x(0)
    prev_base = jnp.int32(0)
    prev_fb = jnp.bool_(False)
    for r in range(ROWS_PER_W):
        row = r0 + r
        b = r % 2
        if r + 1 < ROWS_PER_W:
            issue_x(r + 1)
        hx[r].wait()
        xrow = xb.at[b]

        smx = []
        for s in range(NSEG):
            def step(t, mxc, _s=s):
                off0 = _s * SEG + t * (L * U)
                xs = [xrow[pl.ds(off0 + u * L, L)] for u in range(U)]
                m01 = jnp.maximum(xs[0], xs[1])
                m23 = jnp.maximum(xs[2], xs[3])
                m45 = jnp.maximum(xs[4], xs[5])
                m67 = jnp.maximum(xs[6], xs[7])
                mg = jnp.maximum(jnp.maximum(m01, m23),
                                 jnp.maximum(m45, m67))
                return jnp.maximum(mxc, mg)

            smx.append(lax.fori_loop(
                0, VPS // U, step,
                jnp.full((L,), -jnp.inf, jnp.float32), unroll=2))

        sms = []
        for s in range(NSEG):
            mc = smx[s][0]
            for j in range(1, L):
                mc = jnp.maximum(mc, smx[s][j])
            sms.append(mc)
        m = sms[0]
        for s in range(1, NSEG):
            m = jnp.maximum(m, sms[s])
        sw = jnp.int32(NSEG - 1)
        for s in range(NSEG - 2, -1, -1):
            sw = jnp.where(sms[s] == m, jnp.int32(s), sw)
        any_spike = m >= 4.0

        mvec = lax.broadcast_in_dim(m, (L,), ())
        seg0 = sw * SEG

        def sstep(t, fnd):
            xv = xrow[pl.ds(seg0 + t * L, L)]
            hit = jnp.logical_and(xv == mvec, fnd == BIG)
            return jnp.where(hit, ii + t * L, fnd)

        fnd = lax.fori_loop(0, VPS, sstep,
                            jnp.full((L,), BIG, jnp.int32), unroll=8)
        win_in = fnd[0]
        for j in range(1, L):
            win_in = jnp.minimum(win_in, fnd[j])
        win = jnp.where(any_spike, seg0 + win_in, jnp.int32(0))
        base = (win // L) * L
        off = win - base

        if r > 0:
            pltpu.make_async_copy(x_hbm.at[row], vconst, semv).wait()

            @pl.when(prev_fb)
            def _():
                lax.fori_loop(0, N // L, refill, 0, unroll=8)

            @pl.when(jnp.logical_not(prev_fb))
            def _(pb=prev_base):
                vconst[pl.ds(pb, L)] = jnp.full((L,), -5.0, jnp.float32)

        @pl.when(any_spike)
        def _():
            vconst[pl.ds(base, L)] = jnp.where(
                ii == off, jnp.float32(0.0), jnp.float32(-5.0))
            pltpu.async_copy(vconst, v_hbm.at[row], semv)

        @pl.when(jnp.logical_not(any_spike))
        def _():
            def vstep(t, _):
                vconst[pl.ds(t * L, L)] = xrow[pl.ds(t * L, L)] * 0.25
                return 0

            lax.fori_loop(0, N // L, vstep, 0, unroll=8)
            pltpu.async_copy(vconst, v_hbm.at[row], semv)

        prev_base = base
        prev_fb = jnp.logical_not(any_spike)

    pltpu.make_async_copy(x_hbm.at[r0], vconst, semv).wait()


@jax.jit
def _lif_hybrid(x):
    f32 = jnp.float32

    def zi_body(x_ref, z_ref, i_ref):
        xv = x_ref[...]
        i_ref[...] = xv * 0.5
        z_ref[...] = jnp.where(xv >= 4.0, jnp.float32(1.0),
                               jnp.float32(0.0))

    z, i = pl.pallas_call(
        zi_body,
        grid=(16,),
        in_specs=[pl.BlockSpec((8, N), lambda g: (g, 0))],
        out_specs=[pl.BlockSpec((8, N), lambda g: (g, 0)),
                   pl.BlockSpec((8, N), lambda g: (g, 0))],
        out_shape=[jax.ShapeDtypeStruct((B, N), f32),
                   jax.ShapeDtypeStruct((B, N), f32)],
    )(x)

    v = functools.partial(
        pl.kernel,
        mesh=plsc.VectorSubcoreMesh(core_axis_name="c", subcore_axis_name="s"),
        out_type=jax.ShapeDtypeStruct((B, N), f32),
        scratch_types=[
            pltpu.VMEM((2, N), f32),
            pltpu.VMEM((N,), f32),
            pltpu.SemaphoreType.DMA((2,)),
            pltpu.SemaphoreType.DMA,
        ],
    )(_v_body)(x)

    return z, v, i


def kernel(x):
    z, new_v, i_new = _lif_hybrid(x)
    return z, new_v, i_new


# speedup vs baseline: 2.6808x; 1.0128x over previous
import functools

import jax
import jax.numpy as jnp
from jax import lax
from jax.experimental import pallas as pl
from jax.experimental.pallas import tpu as pltpu
from jax.experimental.pallas import tpu_sc as plsc

B = 128
N = 32768
NC = 2
NS = 16
NW = NC * NS
ROWS_PER_W = B // NW
L = 16
U = 8
NSEG = 4
SEG = N // NSEG
VPS = SEG // L
FBC = 8192
BIG = 2**31 - 1


def _v_body(x_hbm, v_hbm, xb, vconst, semx, semv):
    wid = lax.axis_index("s") * NC + lax.axis_index("c")
    r0 = wid * ROWS_PER_W
    ii = lax.iota(jnp.int32, L)

    def refill(j, _):
        vconst[pl.ds(j * L, L)] = jnp.full((L,), -5.0, jnp.float32)
        return 0

    lax.fori_loop(0, N // L, refill, 0, unroll=8)

    hx = {}

    def issue_x(r):
        hx[r] = pltpu.async_copy(x_hbm.at[r0 + r], xb.at[r % 2],
                                 semx.at[r % 2])

    issue_x(0)
    prev_base = jnp.int32(0)
    prev_fb = jnp.bool_(False)
    for r in range(ROWS_PER_W):
        row = r0 + r
        b = r % 2
        if r + 1 < ROWS_PER_W:
            issue_x(r + 1)
        hx[r].wait()
        xrow = xb.at[b]

        smx = []
        for s in range(NSEG):
            def step(t, mxc, _s=s):
                off0 = _s * SEG + t * (L * U)
                xs = [xrow[pl.ds(off0 + u * L, L)] for u in range(U)]
                m01 = jnp.maximum(xs[0], xs[1])
                m23 = jnp.maximum(xs[2], xs[3])
                m45 = jnp.maximum(xs[4], xs[5])
                m67 = jnp.maximum(xs[6], xs[7])
                mg = jnp.maximum(jnp.maximum(m01, m23),
                                 jnp.maximum(m45, m67))
                return jnp.maximum(mxc, mg)

            smx.append(lax.fori_loop(
                0, VPS // U, step,
                jnp.full((L,), -jnp.inf, jnp.float32), unroll=2))

        sms = []
        for s in range(NSEG):
            mc = smx[s][0]
            for j in range(1, L):
                mc = jnp.maximum(mc, smx[s][j])
            sms.append(mc)
        m = sms[0]
        for s in range(1, NSEG):
            m = jnp.maximum(m, sms[s])
        sw = jnp.int32(NSEG - 1)
        for s in range(NSEG - 2, -1, -1):
            sw = jnp.where(sms[s] == m, jnp.int32(s), sw)
        any_spike = m >= 4.0

        mvec = lax.broadcast_in_dim(m, (L,), ())
        seg0 = sw * SEG

        def sstep(t, fnd):
            xv = xrow[pl.ds(seg0 + t * L, L)]
            hit = jnp.logical_and(xv == mvec, fnd == BIG)
            return jnp.where(hit, ii + t * L, fnd)

        fnd = lax.fori_loop(0, VPS, sstep,
                            jnp.full((L,), BIG, jnp.int32), unroll=8)
        win_in = fnd[0]
        for j in range(1, L):
            win_in = jnp.minimum(win_in, fnd[j])
        win = jnp.where(any_spike, seg0 + win_in, jnp.int32(0))
        base = (win // L) * L
        off = win - base

        if r > 0:
            pltpu.make_async_copy(x_hbm.at[row], vconst, semv).wait()

            @pl.when(prev_fb)
            def _():
                lax.fori_loop(0, N // L, refill, 0, unroll=8)

            @pl.when(jnp.logical_not(prev_fb))
            def _(pb=prev_base):
                vconst[pl.ds(pb, L)] = jnp.full((L,), -5.0, jnp.float32)

        @pl.when(any_spike)
        def _():
            vconst[pl.ds(base, L)] = jnp.where(
                ii == off, jnp.float32(0.0), jnp.float32(-5.0))
            pltpu.async_copy(vconst, v_hbm.at[row], semv)

        @pl.when(jnp.logical_not(any_spike))
        def _():
            def vstep(t, _):
                vconst[pl.ds(t * L, L)] = xrow[pl.ds(t * L, L)] * 0.25
                return 0

            lax.fori_loop(0, N // L, vstep, 0, unroll=8)
            pltpu.async_copy(vconst, v_hbm.at[row], semv)

        prev_base = base
        prev_fb = jnp.logical_not(any_spike)

    pltpu.make_async_copy(x_hbm.at[r0], vconst, semv).wait()


@jax.jit
def _lif_hybrid(x):
    f32 = jnp.float32

    def zi_body(x_ref, z_ref, i_ref):
        xv = x_ref[...]
        i_ref[...] = xv * 0.5
        z_ref[...] = jnp.where(xv >= 4.0, jnp.float32(1.0),
                               jnp.float32(0.0))

    TR = 32
    z, i = pl.pallas_call(
        zi_body,
        grid=(B // TR,),
        in_specs=[pl.BlockSpec((TR, N), lambda g: (g, 0))],
        out_specs=[pl.BlockSpec((TR, N), lambda g: (g, 0)),
                   pl.BlockSpec((TR, N), lambda g: (g, 0))],
        out_shape=[jax.ShapeDtypeStruct((B, N), f32),
                   jax.ShapeDtypeStruct((B, N), f32)],
    )(x)

    v = functools.partial(
        pl.kernel,
        mesh=plsc.VectorSubcoreMesh(core_axis_name="c", subcore_axis_name="s"),
        out_type=jax.ShapeDtypeStruct((B, N), f32),
        scratch_types=[
            pltpu.VMEM((2, N), f32),
            pltpu.VMEM((N,), f32),
            pltpu.SemaphoreType.DMA((2,)),
            pltpu.SemaphoreType.DMA,
        ],
    )(_v_body)(x)

    return z, v, i


def kernel(x):
    z, new_v, i_new = _lif_hybrid(x)
    return z, new_v, i_new
